# Initial kernel scaffold; baseline (speedup 1.0000x reference)
import jax
import jax.numpy as jnp
from jax.experimental import pallas as pl


def kernel(logits):
    raise NotImplementedError("write your pallas kernel here")


# speedup vs baseline: 1.1180x; 1.1180x over previous
import jax
import jax.numpy as jnp
from jax.experimental import pallas as pl

_ROWS = 8


def _hardmax_block(x_ref, o_ref):
    x = x_ref[...]
    m = jnp.max(x, axis=1, keepdims=True)
    col = jax.lax.broadcasted_iota(jnp.int32, x.shape, 1)
    cand = jnp.where(x == m, col, jnp.iinfo(jnp.int32).max)
    idx = jnp.min(cand, axis=1, keepdims=True)
    inf = jnp.float32(jnp.inf)
    o_ref[...] = jnp.where(col == idx, inf, -inf)


def kernel(logits):
    n, d = logits.shape
    return pl.pallas_call(
        _hardmax_block,
        grid=(n // _ROWS,),
        in_specs=[pl.BlockSpec((_ROWS, d), lambda i: (i, 0))],
        out_specs=pl.BlockSpec((_ROWS, d), lambda i: (i, 0)),
        out_shape=jax.ShapeDtypeStruct((n, d), jnp.float32),
    )(logits)


# speedup vs baseline: 1.5476x; 1.3842x over previous
import jax
import jax.numpy as jnp
from jax.experimental import pallas as pl

_ROWS = 16


def _hardmax_block(x_ref, o_ref):
    x = x_ref[...]
    m = jnp.max(x, axis=1, keepdims=True)
    col = jax.lax.broadcasted_iota(jnp.int32, x.shape, 1)
    cand = jnp.where(x == m, col, jnp.iinfo(jnp.int32).max)
    idx = jnp.min(cand, axis=1, keepdims=True)
    inf = jnp.float32(jnp.inf)
    o_ref[...] = jnp.where(col == idx, inf, -inf)


def kernel(logits):
    n, d = logits.shape
    return pl.pallas_call(
        _hardmax_block,
        grid=(n // _ROWS,),
        in_specs=[pl.BlockSpec((_ROWS, d), lambda i: (i, 0))],
        out_specs=pl.BlockSpec((_ROWS, d), lambda i: (i, 0)),
        out_shape=jax.ShapeDtypeStruct((n, d), jnp.float32),
    )(logits)


# speedup vs baseline: 1.7457x; 1.1280x over previous
import jax
import jax.numpy as jnp
from jax.experimental import pallas as pl

_ROWS = 32


def _hardmax_block(x_ref, o_ref):
    x = x_ref[...]
    m = jnp.max(x, axis=1, keepdims=True)
    col = jax.lax.broadcasted_iota(jnp.int32, x.shape, 1)
    cand = jnp.where(x == m, col, jnp.iinfo(jnp.int32).max)
    idx = jnp.min(cand, axis=1, keepdims=True)
    inf = jnp.float32(jnp.inf)
    o_ref[...] = jnp.where(col == idx, inf, -inf)


def kernel(logits):
    n, d = logits.shape
    return pl.pallas_call(
        _hardmax_block,
        grid=(n // _ROWS,),
        in_specs=[pl.BlockSpec((_ROWS, d), lambda i: (i, 0))],
        out_specs=pl.BlockSpec((_ROWS, d), lambda i: (i, 0)),
        out_shape=jax.ShapeDtypeStruct((n, d), jnp.float32),
    )(logits)


# speedup vs baseline: 2.0324x; 1.1642x over previous
import jax
import jax.numpy as jnp
from jax.experimental import pallas as pl

_ROWS = 64


def _hardmax_block(x_ref, o_ref):
    x = x_ref[...]
    m = jnp.max(x, axis=1, keepdims=True)
    col = jax.lax.broadcasted_iota(jnp.int32, x.shape, 1)
    cand = jnp.where(x == m, col, jnp.iinfo(jnp.int32).max)
    idx = jnp.min(cand, axis=1, keepdims=True)
    inf = jnp.float32(jnp.inf)
    o_ref[...] = jnp.where(col == idx, inf, -inf)


def kernel(logits):
    n, d = logits.shape
    return pl.pallas_call(
        _hardmax_block,
        grid=(n // _ROWS,),
        in_specs=[pl.BlockSpec((_ROWS, d), lambda i: (i, 0))],
        out_specs=pl.BlockSpec((_ROWS, d), lambda i: (i, 0)),
        out_shape=jax.ShapeDtypeStruct((n, d), jnp.float32),
    )(logits)
